# baseline (speedup 1.0000x reference)
import functools

import jax
import jax.numpy as jnp
from jax import lax
from jax.experimental import pallas as pl
from jax.experimental.pallas import tpu as pltpu
from jax.experimental.pallas import tpu_sc as plsc

NC, NS, L = 2, 16, 16
NW = NC * NS
B = 16384
BPW = B // NW
HALF = BPW // 2
E = 64


def _sc_gather(uidx, iidx, ue_gmf, ie_gmf, ue_mlp, ie_mlp):
    mesh = plsc.VectorSubcoreMesh(
        core_axis_name="c", subcore_axis_name="s", num_cores=NC, num_subcores=NS
    )

    @functools.partial(
        pl.kernel,
        out_type=(
            jax.ShapeDtypeStruct((B, 2 * E), jnp.float32),
            jax.ShapeDtypeStruct((B, 2 * E), jnp.float32),
        ),
        mesh=mesh,
        scratch_types=[
            pltpu.VMEM((BPW,), jnp.int32),
            pltpu.VMEM((BPW,), jnp.int32),
            pltpu.VMEM((HALF, 2 * E), jnp.float32),
            pltpu.VMEM((HALF, 2 * E), jnp.float32),
            pltpu.SemaphoreType.DMA,
            pltpu.SemaphoreType.DMA,
            pltpu.SemaphoreType.DMA,
            pltpu.SemaphoreType.DMA,
        ],
    )
    def k(uidx_hbm, iidx_hbm, ueg, ieg, uem, iem,
          mlp_out, gmf_out, uidx_v, iidx_v, mlp_v, gmf_v,
          sem, sem2, sem3, sem4):
        wid = lax.axis_index("s") * NC + lax.axis_index("c")
        base = wid * BPW

        pltpu.sync_copy(uidx_hbm.at[pl.ds(base, BPW)], uidx_v)
        pltpu.sync_copy(iidx_hbm.at[pl.ds(base, BPW)], iidx_v)

        for h in range(2):
            off = h * HALF

            def body(j, _):
                uvec = uidx_v[pl.ds(off + 16 * j, 16)]
                ivec = iidx_v[pl.ds(off + 16 * j, 16)]
                for kk in range(16):
                    u = uvec[kk]
                    i = ivec[kk]
                    d = 16 * j + kk
                    pltpu.async_copy(uem.at[u], mlp_v.at[d, pl.ds(0, E)], sem)
                    pltpu.async_copy(iem.at[i], mlp_v.at[d, pl.ds(E, E)], sem2)
                    pltpu.async_copy(ueg.at[u], gmf_v.at[d, pl.ds(0, E)], sem3)
                    pltpu.async_copy(ieg.at[i], gmf_v.at[d, pl.ds(E, E)], sem4)
                return 0

            lax.fori_loop(0, HALF // 16, body, 0)
            pltpu.make_async_copy(
                mlp_out.at[pl.ds(0, HALF // 2)],
                mlp_v.at[pl.ds(0, HALF // 2)], sem).wait()
            pltpu.make_async_copy(
                mlp_out.at[pl.ds(0, HALF // 2)],
                mlp_v.at[pl.ds(0, HALF // 2)], sem2).wait()
            pltpu.make_async_copy(
                gmf_out.at[pl.ds(0, HALF // 2)],
                gmf_v.at[pl.ds(0, HALF // 2)], sem3).wait()
            pltpu.make_async_copy(
                gmf_out.at[pl.ds(0, HALF // 2)],
                gmf_v.at[pl.ds(0, HALF // 2)], sem4).wait()
            pltpu.sync_copy(mlp_v, mlp_out.at[pl.ds(base + off, HALF)])
            pltpu.sync_copy(gmf_v, gmf_out.at[pl.ds(base + off, HALF)])

    return k(uidx, iidx, ue_gmf, ie_gmf, ue_mlp, ie_mlp)


BLK = 2048


def _tc_tower(mlp_in, gmf_in, w1t, b1r, w2t, b2r, w3t, b3r, wpg_r, wpm_r, bp_r):
    def body(x_ref, g_ref, w1_ref, b1_ref, w2_ref, b2_ref, w3_ref, b3_ref,
             wpg_ref, wpm_ref, bp_ref, o_ref):
        g = g_ref[...]
        gdot = jnp.sum(g[:, :E] * g[:, E:] * wpg_ref[...], axis=1)
        h = jnp.dot(x_ref[...], w1_ref[...], preferred_element_type=jnp.float32)
        h = jnp.maximum(h + b1_ref[...], 0.0)
        h = jnp.maximum(
            jnp.dot(h, w2_ref[...], preferred_element_type=jnp.float32)
            + b2_ref[...], 0.0)
        h = jnp.maximum(
            jnp.dot(h, w3_ref[...], preferred_element_type=jnp.float32)
            + b3_ref[...], 0.0)
        o_ref[...] = gdot + jnp.sum(h * wpm_ref[...], axis=1) + bp_ref[0, 0]

    full = lambda r, c: pl.BlockSpec((r, c), lambda i: (0, 0))
    out = pl.pallas_call(
        body,
        grid=(B // BLK,),
        in_specs=[
            pl.BlockSpec((BLK, 2 * E), lambda i: (i, 0)),
            pl.BlockSpec((BLK, 2 * E), lambda i: (i, 0)),
            full(2 * E, E), full(1, E),
            full(E, 32), full(1, 32),
            full(32, 16), full(1, 16),
            full(1, E), full(1, 16), full(1, 1),
        ],
        out_specs=pl.BlockSpec((BLK,), lambda i: (i,)),
        out_shape=jax.ShapeDtypeStruct((B,), jnp.float32),
    )(mlp_in, gmf_in, w1t, b1r, w2t, b2r, w3t, b3r, wpg_r, wpm_r, bp_r)
    return out


def kernel(user_indices, item_indices, ue_gmf, ie_gmf, ue_mlp, ie_mlp,
           W1, b1, W2, b2, W3, b3, Wp, bp):
    uidx = user_indices.astype(jnp.int32)
    iidx = item_indices.astype(jnp.int32)

    mlp_in, gmf_in = _sc_gather(uidx, iidx, ue_gmf, ie_gmf, ue_mlp, ie_mlp)

    return _tc_tower(mlp_in, gmf_in,
                     W1.T, b1.reshape(1, E),
                     W2.T, b2.reshape(1, 32),
                     W3.T, b3.reshape(1, 16),
                     Wp[0, :E].reshape(1, E), Wp[0, E:].reshape(1, L),
                     bp.reshape(1, 1))


# speedup vs baseline: 1.1728x; 1.1728x over previous
import functools

import jax
import jax.numpy as jnp
from jax import lax
from jax.experimental import pallas as pl
from jax.experimental.pallas import tpu as pltpu
from jax.experimental.pallas import tpu_sc as plsc

NC, NS, L = 2, 16, 16
NW = NC * NS
B = 16384
BPW = B // NW
E = 64
CHUNK = 128
NROWS = 1000000
BIG = 2 ** 30


def _sc_gather(su, pu, si, pi, uegT, iegT, uemT, iemT):
    mesh = plsc.VectorSubcoreMesh(
        core_axis_name="c", subcore_axis_name="s", num_cores=NC, num_subcores=NS
    )

    @functools.partial(
        pl.kernel,
        out_type=(
            jax.ShapeDtypeStruct((B, 2 * E), jnp.float32),
            jax.ShapeDtypeStruct((B, 2 * E), jnp.float32),
        ),
        mesh=mesh,
        compiler_params=pltpu.CompilerParams(
            disable_bounds_checks=True, needs_layout_passes=False),
        scratch_types=[
            pltpu.VMEM((BPW,), jnp.int32),
            pltpu.VMEM((4, 128), jnp.int32),
            pltpu.VMEM((E, CHUNK), jnp.float32),
            pltpu.VMEM((E, CHUNK), jnp.float32),
            pltpu.VMEM((BPW, 2 * E), jnp.float32),
            pltpu.SemaphoreType.DMA,
        ],
    )
    def k(su_hbm, pu_hbm, si_hbm, pi_hbm, ueg, ieg, uem, iem,
          u_out, i_out, sidx_v, pos_v, ch_m, ch_g, st_v, sem):
        wid = lax.axis_index("s") * NC + lax.axis_index("c")
        base = wid * BPW
        lanes = lax.iota(jnp.int32, 16)

        def run_pass(s_hbm, p_hbm, t_mlp, t_gmf, out_hbm):
            pltpu.sync_copy(s_hbm.at[pl.ds(base, BPW)], sidx_v)
            pltpu.sync_copy(p_hbm.at[wid], pos_v)

            r_first = plsc.load_gather(sidx_v, [jnp.zeros((16,), jnp.int32)])[0]
            r_last = plsc.load_gather(
                sidx_v, [jnp.full((16,), BPW - 1, jnp.int32)])[0]
            c_lo = r_first // CHUNK
            c_hi = r_last // CHUNK

            def chunk_body(c, ptr):
                start = c * CHUNK
                pltpu.sync_copy(t_mlp.at[:, pl.ds(start, CHUNK)], ch_m)
                pltpu.sync_copy(t_gmf.at[:, pl.ds(start, CHUNK)], ch_g)

                def cond(carry):
                    p, cur = carry
                    return jnp.logical_and(p < BPW, cur // CHUNK == c)

                def body(carry):
                    p, cur = carry
                    l = jnp.full((16,), cur - start, jnp.int32)
                    for c4 in range(4):
                        e_idx = lanes + 16 * c4
                        st_v[p, pl.ds(16 * c4, 16)] = plsc.load_gather(
                            ch_m, [e_idx, l])
                        st_v[p, pl.ds(E + 16 * c4, 16)] = plsc.load_gather(
                            ch_g, [e_idx, l])
                    p1 = p + 1
                    nxt = plsc.load_gather(
                        sidx_v, [jnp.full((16,), 0, jnp.int32)
                                 + jnp.minimum(p1, BPW - 1)])[0]
                    nxt = jnp.where(p1 < BPW, nxt, BIG)
                    return p1, nxt

                cur0 = plsc.load_gather(
                    sidx_v, [jnp.full((16,), 0, jnp.int32)
                             + jnp.minimum(ptr, BPW - 1)])[0]
                cur0 = jnp.where(ptr < BPW, cur0, BIG)
                ptr, _ = lax.while_loop(cond, body, (ptr, cur0))
                return ptr

            lax.fori_loop(c_lo, c_hi + 1, chunk_body, jnp.int32(0))

            cps = []
            for j in range(4):
                cps.append(pltpu.async_copy(
                    st_v.at[pl.ds(128 * j, 128)],
                    out_hbm.at[pos_v.at[j]], sem))
            for cp in cps:
                cp.wait()

        run_pass(su_hbm, pu_hbm, uem, ueg, u_out)
        run_pass(si_hbm, pi_hbm, iem, ieg, i_out)

    return k(su, pu, si, pi, uegT, iegT, uemT, iemT)


BLK = 2048


def _tc_tower(u_in, i_in, w1at, w1bt, b1r, w2t, b2r, w3t, b3r,
              wpg_r, wpm_r, bp_r):
    def body(u_ref, i_ref, w1a_ref, w1b_ref, b1_ref, w2_ref, b2_ref,
             w3_ref, b3_ref, wpg_ref, wpm_ref, bp_ref, o_ref):
        u = u_ref[...]
        i = i_ref[...]
        gdot = jnp.sum(u[:, E:] * i[:, E:] * wpg_ref[...], axis=1)
        h = jnp.dot(u[:, :E], w1a_ref[...], preferred_element_type=jnp.float32)
        h = h + jnp.dot(i[:, :E], w1b_ref[...],
                        preferred_element_type=jnp.float32)
        h = jnp.maximum(h + b1_ref[...], 0.0)
        h = jnp.maximum(
            jnp.dot(h, w2_ref[...], preferred_element_type=jnp.float32)
            + b2_ref[...], 0.0)
        h = jnp.maximum(
            jnp.dot(h, w3_ref[...], preferred_element_type=jnp.float32)
            + b3_ref[...], 0.0)
        o_ref[...] = gdot + jnp.sum(h * wpm_ref[...], axis=1) + bp_ref[0, 0]

    full = lambda r, c: pl.BlockSpec((r, c), lambda i: (0, 0))
    out = pl.pallas_call(
        body,
        grid=(B // BLK,),
        in_specs=[
            pl.BlockSpec((BLK, 2 * E), lambda i: (i, 0)),
            pl.BlockSpec((BLK, 2 * E), lambda i: (i, 0)),
            full(E, E), full(E, E), full(1, E),
            full(E, 32), full(1, 32),
            full(32, 16), full(1, 16),
            full(1, E), full(1, 16), full(1, 1),
        ],
        out_specs=pl.BlockSpec((BLK,), lambda i: (i,)),
        out_shape=jax.ShapeDtypeStruct((B,), jnp.float32),
    )(u_in, i_in, w1at, w1bt, b1r, w2t, b2r, w3t, b3r, wpg_r, wpm_r, bp_r)
    return out


def kernel(user_indices, item_indices, ue_gmf, ie_gmf, ue_mlp, ie_mlp,
           W1, b1, W2, b2, W3, b3, Wp, bp):
    uidx = user_indices.astype(jnp.int32)
    iidx = item_indices.astype(jnp.int32)
    perm_u = jnp.argsort(uidx).astype(jnp.int32)
    perm_i = jnp.argsort(iidx).astype(jnp.int32)
    su = jnp.take(uidx, perm_u)
    si = jnp.take(iidx, perm_i)
    pu = perm_u.reshape(NW, 4, 128)
    pi = perm_i.reshape(NW, 4, 128)

    u_in, i_in = _sc_gather(su, pu, si, pi,
                            ue_gmf.T, ie_gmf.T, ue_mlp.T, ie_mlp.T)

    return _tc_tower(u_in, i_in,
                     W1[:, :E].T, W1[:, E:].T, b1.reshape(1, E),
                     W2.T, b2.reshape(1, 32),
                     W3.T, b3.reshape(1, 16),
                     Wp[0, :E].reshape(1, E), Wp[0, E:].reshape(1, L),
                     bp.reshape(1, 1))


# speedup vs baseline: 2.3003x; 1.9614x over previous
import functools

import jax
import jax.numpy as jnp
from jax import lax
from jax.experimental import pallas as pl
from jax.experimental.pallas import tpu as pltpu
from jax.experimental.pallas import tpu_sc as plsc

NC, NS, L = 2, 16, 16
NW = NC * NS
B = 16384
BPW = B // NW
HALF = BPW // 2
E = 64
CHUNK = 256
NROWS = 1000000
MAXSTART = 999808
BIG = 2 ** 30


def _sc_gather(su, pu, si, pi, uegT, iegT, uemT, iemT):
    mesh = plsc.VectorSubcoreMesh(
        core_axis_name="c", subcore_axis_name="s", num_cores=NC, num_subcores=NS
    )

    @functools.partial(
        pl.kernel,
        out_type=(
            jax.ShapeDtypeStruct((B, 2 * E), jnp.float32),
            jax.ShapeDtypeStruct((B, 2 * E), jnp.float32),
        ),
        mesh=mesh,
        compiler_params=pltpu.CompilerParams(
            disable_bounds_checks=True, needs_layout_passes=False),
        scratch_types=[
            pltpu.VMEM((BPW,), jnp.int32),
            pltpu.VMEM((4, 128), jnp.int32),
            pltpu.VMEM((E, CHUNK), jnp.float32),
            pltpu.VMEM((E, CHUNK), jnp.float32),
            pltpu.VMEM((E, CHUNK), jnp.float32),
            pltpu.VMEM((E, CHUNK), jnp.float32),
            pltpu.VMEM((HALF, 2 * E), jnp.float32),
            pltpu.SemaphoreType.DMA,
            pltpu.SemaphoreType.DMA,
            pltpu.SemaphoreType.DMA,
        ],
    )
    def k(su_hbm, pu_hbm, si_hbm, pi_hbm, ueg, ieg, uem, iem,
          u_out, i_out, sidx_v, pos_v, m0, m1, g0, g1, st_v,
          sem0, sem1, semo):
        wid = lax.axis_index("s") * NC + lax.axis_index("c")
        base = wid * BPW
        lanes = lax.iota(jnp.int32, 16)

        def read_idx(p):
            v = plsc.load_gather(
                sidx_v, [jnp.full((16,), 0, jnp.int32)
                         + jnp.minimum(p, BPW - 1)])[0]
            return jnp.where(p < BPW, v, BIG)

        def run_half(h, s_hbm, p_hbm, t_mlp, t_gmf, out_hbm, do_load):
            if do_load:
                pltpu.sync_copy(s_hbm.at[pl.ds(base, BPW)], sidx_v)
                pltpu.sync_copy(p_hbm.at[wid], pos_v)

            p_lo = h * HALF
            r_first = read_idx(jnp.int32(p_lo))
            r_last = read_idx(jnp.int32(p_lo + HALF - 1))
            c_lo = r_first // CHUNK
            nch = r_last // CHUNK - c_lo + 1

            def issue(t, mbuf, gbuf, sem):
                start = jnp.minimum((c_lo + t) * CHUNK, MAXSTART)
                pltpu.async_copy(t_mlp.at[:, pl.ds(start, CHUNK)], mbuf, sem)
                pltpu.async_copy(t_gmf.at[:, pl.ds(start, CHUNK)], gbuf, sem)

            def drain(mbuf, gbuf, sem):
                pltpu.make_async_copy(
                    t_mlp.at[:, pl.ds(0, CHUNK)], mbuf, sem).wait()
                pltpu.make_async_copy(
                    t_gmf.at[:, pl.ds(0, CHUNK)], gbuf, sem).wait()

            def extract(t, mbuf, gbuf, ptr):
                c = c_lo + t
                start = jnp.minimum(c * CHUNK, MAXSTART)

                def cond(carry):
                    p, cur = carry
                    return jnp.logical_and(p < p_lo + HALF, cur // CHUNK == c)

                def body(carry):
                    p, cur = carry
                    l = jnp.full((16,), cur - start, jnp.int32)
                    d = p - p_lo
                    for c4 in range(4):
                        e_idx = lanes + 16 * c4
                        st_v[d, pl.ds(16 * c4, 16)] = plsc.load_gather(
                            mbuf, [e_idx, l])
                        st_v[d, pl.ds(E + 16 * c4, 16)] = plsc.load_gather(
                            gbuf, [e_idx, l])
                    p1 = p + 1
                    return p1, read_idx(p1)

                ptr, _ = lax.while_loop(cond, body, (ptr, read_idx(ptr)))
                return ptr

            issue(jnp.int32(0), m0, g0, sem0)

            def pair_body(tp, ptr):
                t0 = 2 * tp
                drain(m0, g0, sem0)
                issue(t0 + 1, m1, g1, sem1)
                ptr = extract(t0, m0, g0, ptr)
                drain(m1, g1, sem1)
                issue(t0 + 2, m0, g0, sem0)
                ptr = extract(t0 + 1, m1, g1, ptr)
                return ptr

            npairs = (nch + 1) // 2
            lax.fori_loop(0, npairs, pair_body, jnp.int32(p_lo))
            drain(m0, g0, sem0)

            cps = []
            for j in range(2):
                cps.append(pltpu.async_copy(
                    st_v.at[pl.ds(128 * j, 128)],
                    out_hbm.at[pos_v.at[2 * h + j]], semo))
            for cp in cps:
                cp.wait()

        run_half(0, su_hbm, pu_hbm, uem, ueg, u_out, True)
        run_half(1, su_hbm, pu_hbm, uem, ueg, u_out, False)
        run_half(0, si_hbm, pi_hbm, iem, ieg, i_out, True)
        run_half(1, si_hbm, pi_hbm, iem, ieg, i_out, False)

    return k(su, pu, si, pi, uegT, iegT, uemT, iemT)


BLK = 2048


def _tc_tower(u_in, i_in, w1at, w1bt, b1r, w2t, b2r, w3t, b3r,
              wpg_r, wpm_r, bp_r):
    def body(u_ref, i_ref, w1a_ref, w1b_ref, b1_ref, w2_ref, b2_ref,
             w3_ref, b3_ref, wpg_ref, wpm_ref, bp_ref, o_ref):
        u = u_ref[...]
        i = i_ref[...]
        gdot = jnp.sum(u[:, E:] * i[:, E:] * wpg_ref[...], axis=1)
        h = jnp.dot(u[:, :E], w1a_ref[...], preferred_element_type=jnp.float32)
        h = h + jnp.dot(i[:, :E], w1b_ref[...],
                        preferred_element_type=jnp.float32)
        h = jnp.maximum(h + b1_ref[...], 0.0)
        h = jnp.maximum(
            jnp.dot(h, w2_ref[...], preferred_element_type=jnp.float32)
            + b2_ref[...], 0.0)
        h = jnp.maximum(
            jnp.dot(h, w3_ref[...], preferred_element_type=jnp.float32)
            + b3_ref[...], 0.0)
        o_ref[...] = gdot + jnp.sum(h * wpm_ref[...], axis=1) + bp_ref[0, 0]

    full = lambda r, c: pl.BlockSpec((r, c), lambda i: (0, 0))
    out = pl.pallas_call(
        body,
        grid=(B // BLK,),
        in_specs=[
            pl.BlockSpec((BLK, 2 * E), lambda i: (i, 0)),
            pl.BlockSpec((BLK, 2 * E), lambda i: (i, 0)),
            full(E, E), full(E, E), full(1, E),
            full(E, 32), full(1, 32),
            full(32, 16), full(1, 16),
            full(1, E), full(1, 16), full(1, 1),
        ],
        out_specs=pl.BlockSpec((BLK,), lambda i: (i,)),
        out_shape=jax.ShapeDtypeStruct((B,), jnp.float32),
    )(u_in, i_in, w1at, w1bt, b1r, w2t, b2r, w3t, b3r, wpg_r, wpm_r, bp_r)
    return out


def kernel(user_indices, item_indices, ue_gmf, ie_gmf, ue_mlp, ie_mlp,
           W1, b1, W2, b2, W3, b3, Wp, bp):
    uidx = user_indices.astype(jnp.int32)
    iidx = item_indices.astype(jnp.int32)
    perm_u = jnp.argsort(uidx).astype(jnp.int32)
    perm_i = jnp.argsort(iidx).astype(jnp.int32)
    su = jnp.take(uidx, perm_u)
    si = jnp.take(iidx, perm_i)
    pu = perm_u.reshape(NW, 4, 128)
    pi = perm_i.reshape(NW, 4, 128)

    u_in, i_in = _sc_gather(su, pu, si, pi,
                            ue_gmf.T, ie_gmf.T, ue_mlp.T, ie_mlp.T)

    return _tc_tower(u_in, i_in,
                     W1[:, :E].T, W1[:, E:].T, b1.reshape(1, E),
                     W2.T, b2.reshape(1, 32),
                     W3.T, b3.reshape(1, 16),
                     Wp[0, :E].reshape(1, E), Wp[0, E:].reshape(1, L),
                     bp.reshape(1, 1))


# speedup vs baseline: 2.3468x; 1.0202x over previous
import functools

import jax
import jax.numpy as jnp
from jax import lax
from jax.experimental import pallas as pl
from jax.experimental.pallas import tpu as pltpu
from jax.experimental.pallas import tpu_sc as plsc

NC, NS, L = 2, 16, 16
NW = NC * NS
B = 16384
BPW = B // NW
HALF = BPW // 2
E = 64
CHUNK = 256
NROWS = 1000000
MAXSTART = 999808
BIG = 2 ** 30


def _sc_gather(su, pu, si, pi, uegT, iegT, uemT, iemT):
    mesh = plsc.VectorSubcoreMesh(
        core_axis_name="c", subcore_axis_name="s", num_cores=NC, num_subcores=NS
    )

    @functools.partial(
        pl.kernel,
        out_type=(
            jax.ShapeDtypeStruct((B, 2 * E), jnp.float32),
            jax.ShapeDtypeStruct((B, 2 * E), jnp.float32),
        ),
        mesh=mesh,
        compiler_params=pltpu.CompilerParams(
            disable_bounds_checks=True, needs_layout_passes=False),
        scratch_types=[
            pltpu.VMEM((BPW,), jnp.int32),
            pltpu.VMEM((4, 128), jnp.int32),
            pltpu.VMEM((E, CHUNK), jnp.float32),
            pltpu.VMEM((E, CHUNK), jnp.float32),
            pltpu.VMEM((E, CHUNK), jnp.float32),
            pltpu.VMEM((E, CHUNK), jnp.float32),
            pltpu.VMEM((HALF, 2 * E), jnp.float32),
            pltpu.SemaphoreType.DMA,
            pltpu.SemaphoreType.DMA,
            pltpu.SemaphoreType.DMA,
        ],
    )
    def k(su_hbm, pu_hbm, si_hbm, pi_hbm, ueg, ieg, uem, iem,
          u_out, i_out, sidx_v, pos_v, m0, m1, g0, g1, st_v,
          sem0, sem1, semo):
        wid = lax.axis_index("s") * NC + lax.axis_index("c")
        base = wid * BPW
        lanes = lax.iota(jnp.int32, 16)

        def read_idx(p):
            v = plsc.load_gather(
                sidx_v, [jnp.full((16,), 0, jnp.int32)
                         + jnp.minimum(p, BPW - 1)])[0]
            return jnp.where(p < BPW, v, BIG)

        def run_half(h, s_hbm, p_hbm, t_mlp, t_gmf, out_hbm, do_load):
            if do_load:
                pltpu.sync_copy(s_hbm.at[pl.ds(base, BPW)], sidx_v)
                pltpu.sync_copy(p_hbm.at[wid], pos_v)

            p_lo = h * HALF
            r_first = read_idx(jnp.int32(p_lo))
            r_last = read_idx(jnp.int32(p_lo + HALF - 1))
            c_lo = r_first // CHUNK
            nch = r_last // CHUNK - c_lo + 1

            def issue(t, mbuf, gbuf, sem):
                start = jnp.minimum((c_lo + t) * CHUNK, MAXSTART)
                pltpu.async_copy(t_mlp.at[:, pl.ds(start, CHUNK)], mbuf, sem)
                pltpu.async_copy(t_gmf.at[:, pl.ds(start, CHUNK)], gbuf, sem)

            def drain(mbuf, gbuf, sem):
                pltpu.make_async_copy(
                    t_mlp.at[:, pl.ds(0, CHUNK)], mbuf, sem).wait()
                pltpu.make_async_copy(
                    t_gmf.at[:, pl.ds(0, CHUNK)], gbuf, sem).wait()

            def extract(t, mbuf, gbuf, ptr):
                c = c_lo + t
                start = jnp.minimum(c * CHUNK, MAXSTART)

                def cond(carry):
                    p, cur = carry
                    return jnp.logical_and(p < p_lo + HALF, cur // CHUNK == c)

                def body(carry):
                    p, cur = carry
                    l = jnp.full((16,), cur - start, jnp.int32)
                    d = p - p_lo
                    for c4 in range(4):
                        e_idx = lanes + 16 * c4
                        st_v[d, pl.ds(16 * c4, 16)] = plsc.load_gather(
                            mbuf, [e_idx, l])
                        st_v[d, pl.ds(E + 16 * c4, 16)] = plsc.load_gather(
                            gbuf, [e_idx, l])
                    p1 = p + 1
                    return p1, read_idx(p1)

                ptr, _ = lax.while_loop(cond, body, (ptr, read_idx(ptr)))
                return ptr

            issue(jnp.int32(0), m0, g0, sem0)

            def pair_body(tp, ptr):
                t0 = 2 * tp
                drain(m0, g0, sem0)
                issue(t0 + 1, m1, g1, sem1)
                ptr = extract(t0, m0, g0, ptr)
                drain(m1, g1, sem1)
                issue(t0 + 2, m0, g0, sem0)
                ptr = extract(t0 + 1, m1, g1, ptr)
                return ptr

            npairs = (nch + 1) // 2
            lax.fori_loop(0, npairs, pair_body, jnp.int32(p_lo))
            drain(m0, g0, sem0)

            cps = []
            for j in range(2):
                cps.append(pltpu.async_copy(
                    st_v.at[pl.ds(128 * j, 128)],
                    out_hbm.at[pos_v.at[2 * h + j]], semo))
            for cp in cps:
                cp.wait()

        run_half(0, su_hbm, pu_hbm, uem, ueg, u_out, True)
        run_half(1, su_hbm, pu_hbm, uem, ueg, u_out, False)
        run_half(0, si_hbm, pi_hbm, iem, ieg, i_out, True)
        run_half(1, si_hbm, pi_hbm, iem, ieg, i_out, False)

    return k(su, pu, si, pi, uegT, iegT, uemT, iemT)


BLK = 2048


def _tc_tower(u_in, i_in, w1at, w1bt, b1r, w2t, b2r, w3t, b3r,
              wpg_r, wpm_r, bp_r):
    def body(u_ref, i_ref, w1a_ref, w1b_ref, b1_ref, w2_ref, b2_ref,
             w3_ref, b3_ref, wpg_ref, wpm_ref, bp_ref, o_ref):
        u = u_ref[...]
        i = i_ref[...]
        gdot = jnp.sum(u[:, E:] * i[:, E:] * wpg_ref[...], axis=1)
        h = jnp.dot(u[:, :E], w1a_ref[...], preferred_element_type=jnp.float32)
        h = h + jnp.dot(i[:, :E], w1b_ref[...],
                        preferred_element_type=jnp.float32)
        h = jnp.maximum(h + b1_ref[...], 0.0)
        h = jnp.maximum(
            jnp.dot(h, w2_ref[...], preferred_element_type=jnp.float32)
            + b2_ref[...], 0.0)
        h = jnp.maximum(
            jnp.dot(h, w3_ref[...], preferred_element_type=jnp.float32)
            + b3_ref[...], 0.0)
        o_ref[...] = gdot + jnp.sum(h * wpm_ref[...], axis=1) + bp_ref[0, 0]

    full = lambda r, c: pl.BlockSpec((r, c), lambda i: (0, 0))
    out = pl.pallas_call(
        body,
        grid=(B // BLK,),
        in_specs=[
            pl.BlockSpec((BLK, 2 * E), lambda i: (i, 0)),
            pl.BlockSpec((BLK, 2 * E), lambda i: (i, 0)),
            full(E, E), full(E, E), full(1, E),
            full(E, 32), full(1, 32),
            full(32, 16), full(1, 16),
            full(1, E), full(1, 16), full(1, 1),
        ],
        out_specs=pl.BlockSpec((BLK,), lambda i: (i,)),
        out_shape=jax.ShapeDtypeStruct((B,), jnp.float32),
    )(u_in, i_in, w1at, w1bt, b1r, w2t, b2r, w3t, b3r, wpg_r, wpm_r, bp_r)
    return out


def kernel(user_indices, item_indices, ue_gmf, ie_gmf, ue_mlp, ie_mlp,
           W1, b1, W2, b2, W3, b3, Wp, bp):
    uidx = user_indices.astype(jnp.int32)
    iidx = item_indices.astype(jnp.int32)
    iota = jnp.arange(B, dtype=jnp.int32)
    su, perm_u = lax.sort((uidx, iota), num_keys=1)
    si, perm_i = lax.sort((iidx, iota), num_keys=1)
    pu = perm_u.reshape(NW, 4, 128)
    pi = perm_i.reshape(NW, 4, 128)

    u_in, i_in = _sc_gather(su, pu, si, pi,
                            ue_gmf.T, ie_gmf.T, ue_mlp.T, ie_mlp.T)

    return _tc_tower(u_in, i_in,
                     W1[:, :E].T, W1[:, E:].T, b1.reshape(1, E),
                     W2.T, b2.reshape(1, 32),
                     W3.T, b3.reshape(1, 16),
                     Wp[0, :E].reshape(1, E), Wp[0, E:].reshape(1, L),
                     bp.reshape(1, 1))


# speedup vs baseline: 2.4722x; 1.0535x over previous
import functools

import jax
import jax.numpy as jnp
from jax import lax
from jax.experimental import pallas as pl
from jax.experimental.pallas import tpu as pltpu
from jax.experimental.pallas import tpu_sc as plsc

NC, NS, L = 2, 16, 16
NW = NC * NS
B = 16384
BPW = B // NW
QTR = BPW // 4
E = 64
CHUNK = 384
NROWS = 1000000
MAXSTART = 999680
BIG = 2 ** 30


def _sc_gather(su, pu, si, pi, uegT, iegT, uemT, iemT):
    mesh = plsc.VectorSubcoreMesh(
        core_axis_name="c", subcore_axis_name="s", num_cores=NC, num_subcores=NS
    )

    @functools.partial(
        pl.kernel,
        out_type=(
            jax.ShapeDtypeStruct((B, 2 * E), jnp.float32),
            jax.ShapeDtypeStruct((B, 2 * E), jnp.float32),
        ),
        mesh=mesh,
        compiler_params=pltpu.CompilerParams(
            disable_bounds_checks=True, needs_layout_passes=False),
        scratch_types=[
            pltpu.VMEM((BPW,), jnp.int32),
            pltpu.VMEM((4, 128), jnp.int32),
            pltpu.VMEM((E, CHUNK), jnp.float32),
            pltpu.VMEM((E, CHUNK), jnp.float32),
            pltpu.VMEM((E, CHUNK), jnp.float32),
            pltpu.VMEM((E, CHUNK), jnp.float32),
            pltpu.VMEM((QTR, 2 * E), jnp.float32),
            pltpu.SemaphoreType.DMA,
            pltpu.SemaphoreType.DMA,
            pltpu.SemaphoreType.DMA,
        ],
    )
    def k(su_hbm, pu_hbm, si_hbm, pi_hbm, ueg, ieg, uem, iem,
          u_out, i_out, sidx_v, pos_v, m0, m1, g0, g1, st_v,
          sem0, sem1, semo):
        wid = lax.axis_index("s") * NC + lax.axis_index("c")
        base = wid * BPW
        lanes = lax.iota(jnp.int32, 16)

        def read_idx(p):
            v = plsc.load_gather(
                sidx_v, [jnp.full((16,), 0, jnp.int32)
                         + jnp.minimum(p, BPW - 1)])[0]
            return jnp.where(p < BPW, v, BIG)

        def run_half(h, s_hbm, p_hbm, t_mlp, t_gmf, out_hbm, do_load):
            if do_load:
                pltpu.sync_copy(s_hbm.at[pl.ds(base, BPW)], sidx_v)
                pltpu.sync_copy(p_hbm.at[wid], pos_v)

            p_lo = h * QTR
            r_first = read_idx(jnp.int32(p_lo))
            r_last = read_idx(jnp.int32(p_lo + QTR - 1))
            c_lo = r_first // CHUNK
            nch = r_last // CHUNK - c_lo + 1

            def issue(t, mbuf, gbuf, sem):
                start = jnp.minimum((c_lo + t) * CHUNK, MAXSTART)
                pltpu.async_copy(t_mlp.at[:, pl.ds(start, CHUNK)], mbuf, sem)
                pltpu.async_copy(t_gmf.at[:, pl.ds(start, CHUNK)], gbuf, sem)

            def drain(mbuf, gbuf, sem):
                pltpu.make_async_copy(
                    t_mlp.at[:, pl.ds(0, CHUNK)], mbuf, sem).wait()
                pltpu.make_async_copy(
                    t_gmf.at[:, pl.ds(0, CHUNK)], gbuf, sem).wait()

            def extract(t, mbuf, gbuf, ptr):
                c = c_lo + t
                start = jnp.minimum(c * CHUNK, MAXSTART)

                def cond(carry):
                    p, cur = carry
                    return jnp.logical_and(p < p_lo + QTR, cur // CHUNK == c)

                def body(carry):
                    p, cur = carry
                    l = jnp.full((16,), cur - start, jnp.int32)
                    d = p - p_lo
                    for c4 in range(4):
                        e_idx = lanes + 16 * c4
                        st_v[d, pl.ds(16 * c4, 16)] = plsc.load_gather(
                            mbuf, [e_idx, l])
                        st_v[d, pl.ds(E + 16 * c4, 16)] = plsc.load_gather(
                            gbuf, [e_idx, l])
                    p1 = p + 1
                    return p1, read_idx(p1)

                ptr, _ = lax.while_loop(cond, body, (ptr, read_idx(ptr)))
                return ptr

            issue(jnp.int32(0), m0, g0, sem0)

            def pair_body(tp, ptr):
                t0 = 2 * tp
                drain(m0, g0, sem0)
                issue(t0 + 1, m1, g1, sem1)
                ptr = extract(t0, m0, g0, ptr)
                drain(m1, g1, sem1)
                issue(t0 + 2, m0, g0, sem0)
                ptr = extract(t0 + 1, m1, g1, ptr)
                return ptr

            npairs = (nch + 1) // 2
            lax.fori_loop(0, npairs, pair_body, jnp.int32(p_lo))
            drain(m0, g0, sem0)

            pltpu.async_copy(st_v, out_hbm.at[pos_v.at[h]], semo).wait()

        for h in range(4):
            run_half(h, su_hbm, pu_hbm, uem, ueg, u_out, h == 0)
        for h in range(4):
            run_half(h, si_hbm, pi_hbm, iem, ieg, i_out, h == 0)

    return k(su, pu, si, pi, uegT, iegT, uemT, iemT)


BLK = 2048


def _tc_tower(u_in, i_in, w1at, w1bt, b1r, w2t, b2r, w3t, b3r,
              wpg_r, wpm_r, bp_r):
    def body(u_ref, i_ref, w1a_ref, w1b_ref, b1_ref, w2_ref, b2_ref,
             w3_ref, b3_ref, wpg_ref, wpm_ref, bp_ref, o_ref):
        u = u_ref[...]
        i = i_ref[...]
        gdot = jnp.sum(u[:, E:] * i[:, E:] * wpg_ref[...], axis=1)
        h = jnp.dot(u[:, :E], w1a_ref[...], preferred_element_type=jnp.float32)
        h = h + jnp.dot(i[:, :E], w1b_ref[...],
                        preferred_element_type=jnp.float32)
        h = jnp.maximum(h + b1_ref[...], 0.0)
        h = jnp.maximum(
            jnp.dot(h, w2_ref[...], preferred_element_type=jnp.float32)
            + b2_ref[...], 0.0)
        h = jnp.maximum(
            jnp.dot(h, w3_ref[...], preferred_element_type=jnp.float32)
            + b3_ref[...], 0.0)
        o_ref[...] = gdot + jnp.sum(h * wpm_ref[...], axis=1) + bp_ref[0, 0]

    full = lambda r, c: pl.BlockSpec((r, c), lambda i: (0, 0))
    out = pl.pallas_call(
        body,
        grid=(B // BLK,),
        in_specs=[
            pl.BlockSpec((BLK, 2 * E), lambda i: (i, 0)),
            pl.BlockSpec((BLK, 2 * E), lambda i: (i, 0)),
            full(E, E), full(E, E), full(1, E),
            full(E, 32), full(1, 32),
            full(32, 16), full(1, 16),
            full(1, E), full(1, 16), full(1, 1),
        ],
        out_specs=pl.BlockSpec((BLK,), lambda i: (i,)),
        out_shape=jax.ShapeDtypeStruct((B,), jnp.float32),
    )(u_in, i_in, w1at, w1bt, b1r, w2t, b2r, w3t, b3r, wpg_r, wpm_r, bp_r)
    return out


def kernel(user_indices, item_indices, ue_gmf, ie_gmf, ue_mlp, ie_mlp,
           W1, b1, W2, b2, W3, b3, Wp, bp):
    uidx = user_indices.astype(jnp.int32)
    iidx = item_indices.astype(jnp.int32)
    iota = jnp.arange(B, dtype=jnp.int32)
    su, perm_u = lax.sort((uidx, iota), num_keys=1)
    si, perm_i = lax.sort((iidx, iota), num_keys=1)
    pu = perm_u.reshape(NW, 4, 128)
    pi = perm_i.reshape(NW, 4, 128)

    u_in, i_in = _sc_gather(su, pu, si, pi,
                            ue_gmf.T, ie_gmf.T, ue_mlp.T, ie_mlp.T)

    return _tc_tower(u_in, i_in,
                     W1[:, :E].T, W1[:, E:].T, b1.reshape(1, E),
                     W2.T, b2.reshape(1, 32),
                     W3.T, b3.reshape(1, 16),
                     Wp[0, :E].reshape(1, E), Wp[0, E:].reshape(1, L),
                     bp.reshape(1, 1))
